# baseline (speedup 1.0000x reference)
import functools

import jax
import jax.numpy as jnp
from jax import lax
from jax.experimental import pallas as pl
from jax.experimental.pallas import tpu as pltpu
from jax.experimental.pallas import tpu_sc as plsc

N = 50000
DIM = 64
OUT = 32
E = 800000
MAX_LOGSTD = 10.0

SENT = 1 << 29
IGN = -1

NPASS = 3
QUART = 8448
QSTRIPE = QUART // 16
HALF = NPASS * QUART
OUTR = 2 * HALF
PCH = 512
PNBUF = 2
TILE_E = 50176
NPHASE = 7
PHASE_E = TILE_E // NPHASE
GROUPS = PHASE_E // (PNBUF * PCH)
E_PAD = 16 * TILE_E

CH = 128
NBUF = 4
DW = 16
ND = 50048
DSTRIPE = ND // 16
DEG_TILE_E = E_PAD // 32
DGROUPS = DEG_TILE_E // (NBUF * CH)

BR = 2000
GRID = N // BR

_mesh = plsc.VectorSubcoreMesh(core_axis_name="c", subcore_axis_name="s")
_sc_params = pltpu.CompilerParams(use_tc_tiling_on_sc=False)


def _fill_idx(sidx, didx, off, base_row, span, sg, dl):
    for j in range(PCH // 16):
        sl = pl.ds(off + j * 16, 16)
        d = didx[sl]
        sv = sidx[sl]
        l = d - base_row
        ok = (l >= 0) & (l < span)
        sg[pl.ds(j * 16, 16)] = jnp.where(ok, sv, IGN)
        dl[pl.ds(j * 16, 16)] = jnp.where(ok, l, IGN)


def _prop_body(y, srcg, dstg, out, sidx, didx, sgs, dls, rows, acc, gsems,
               ssems):
    c = lax.axis_index("c")
    s = lax.axis_index("s")

    def quarter(q, _):
        base_row = c * HALF + q * QUART
        pltpu.sync_copy(y.at[pl.ds(base_row + s * QSTRIPE, QSTRIPE)],
                        acc.at[pl.ds(s * QSTRIPE, QSTRIPE)])
        plsc.subcore_barrier()

        def phase_body(ph, _):
            eoff = s * TILE_E + ph * PHASE_E
            pltpu.sync_copy(srcg.at[pl.ds(eoff, PHASE_E)], sidx)
            pltpu.sync_copy(dstg.at[pl.ds(eoff, PHASE_E)], didx)

            def group(g, _):
                for i in range(PNBUF):
                    def drain(i=i):
                        pltpu.make_async_copy(
                            rows[i],
                            acc.at[plsc.Indices(dls[i], ignored_value=IGN)],
                            ssems[i]).wait()
                    pl.when((ph > 0) | (g > 0))(drain)
                    _fill_idx(sidx, didx, (g * PNBUF + i) * PCH, base_row,
                              QUART, sgs[i], dls[i])
                    pltpu.async_copy(
                        y.at[plsc.Indices(sgs[i], ignored_value=IGN)],
                        rows[i], gsems[i])
                for i in range(PNBUF):
                    pltpu.make_async_copy(
                        y.at[plsc.Indices(sgs[i], ignored_value=IGN)],
                        rows[i], gsems[i]).wait()
                    pltpu.async_copy(
                        rows[i],
                        acc.at[plsc.Indices(dls[i], ignored_value=IGN)],
                        ssems[i], add=True)
                return 0

            lax.fori_loop(0, GROUPS, group, 0)
            return 0

        lax.fori_loop(0, NPHASE, phase_body, 0)

        for i in range(PNBUF):
            pltpu.make_async_copy(
                rows[i], acc.at[plsc.Indices(dls[i], ignored_value=IGN)],
                ssems[i]).wait()
        plsc.subcore_barrier()
        pltpu.sync_copy(acc.at[pl.ds(s * QSTRIPE, QSTRIPE)],
                        out.at[pl.ds(base_row + s * QSTRIPE, QSTRIPE)])
        plsc.subcore_barrier()
        return 0

    lax.fori_loop(0, NPASS, quarter, 0)


_prop_call = pl.kernel(
    _prop_body,
    out_type=jax.ShapeDtypeStruct((OUTR, DIM), jnp.float32),
    mesh=_mesh,
    scratch_types=[
        pltpu.VMEM((PHASE_E,), jnp.int32),
        pltpu.VMEM((PHASE_E,), jnp.int32),
        [pltpu.VMEM((PCH,), jnp.int32) for _ in range(PNBUF)],
        [pltpu.VMEM((PCH,), jnp.int32) for _ in range(PNBUF)],
        [pltpu.VMEM((PCH, DIM), jnp.float32) for _ in range(PNBUF)],
        pltpu.VMEM_SHARED((QUART, DIM), jnp.float32),
        [pltpu.SemaphoreType.DMA for _ in range(PNBUF)],
        [pltpu.SemaphoreType.DMA for _ in range(PNBUF)],
    ],
    compiler_params=_sc_params,
)


def _deg_body(dstg, ones_h, zeros_h, out, didx, obuf, dls, dacc, ssems):
    c = lax.axis_index("c")
    s = lax.axis_index("s")
    w = s * 2 + c

    pltpu.sync_copy(zeros_h, dacc.at[pl.ds(s * DSTRIPE, DSTRIPE)])
    pltpu.sync_copy(ones_h, obuf)
    pltpu.sync_copy(dstg.at[pl.ds(w * DEG_TILE_E, DEG_TILE_E)], didx)
    plsc.subcore_barrier()

    def group(g, _):
        for i in range(NBUF):
            def drain(i=i):
                pltpu.make_async_copy(
                    obuf, dacc.at[plsc.Indices(dls[i], ignored_value=IGN)],
                    ssems[i]).wait()
            pl.when(g > 0)(drain)
            off = (g * NBUF + i) * CH
            for j in range(CH // 16):
                d = didx[pl.ds(off + j * 16, 16)]
                dls[i][pl.ds(j * 16, 16)] = jnp.where(d < N, d, IGN)
            pltpu.async_copy(
                obuf, dacc.at[plsc.Indices(dls[i], ignored_value=IGN)],
                ssems[i], add=True)
        return 0

    lax.fori_loop(0, DGROUPS, group, 0)
    for i in range(NBUF):
        pltpu.make_async_copy(
            obuf, dacc.at[plsc.Indices(dls[i], ignored_value=IGN)],
            ssems[i]).wait()
    plsc.subcore_barrier()
    pltpu.sync_copy(dacc.at[pl.ds(s * DSTRIPE, DSTRIPE)],
                    out.at[pl.ds(c * ND + s * DSTRIPE, DSTRIPE)])


_deg_call = pl.kernel(
    _deg_body,
    out_type=jax.ShapeDtypeStruct((2 * ND, DW), jnp.float32),
    mesh=_mesh,
    scratch_types=[
        pltpu.VMEM((DEG_TILE_E,), jnp.int32),
        pltpu.VMEM((CH, DW), jnp.float32),
        [pltpu.VMEM((CH,), jnp.int32) for _ in range(NBUF)],
        pltpu.VMEM_SHARED((ND, DW), jnp.float32),
        [pltpu.SemaphoreType.DMA for _ in range(NBUF)],
    ],
    compiler_params=_sc_params,
)


def _dinv(d0_ref, d1_ref):
    deg = d0_ref[:, 0:1] + d1_ref[:, 0:1] + 1.0
    return lax.rsqrt(deg)


def _scale_in_body(emb_ref, d0_ref, d1_ref, y_ref):
    y_ref[...] = emb_ref[...] * _dinv(d0_ref, d1_ref)


def _hidden_body(s0_ref, d0_ref, d1_ref, w1_ref, b1_ref, y1_ref):
    dinv = _dinv(d0_ref, d1_ref)
    xh = s0_ref[...] * dinv
    h = jnp.maximum(
        jnp.dot(xh, w1_ref[...], preferred_element_type=jnp.float32)
        + b1_ref[...], 0.0)
    y1_ref[...] = h * dinv


def _out_body(s1_ref, d0_ref, d1_ref, wmu_ref, bmu_ref, wls_ref, bls_ref,
              eps_ref, z_ref):
    g = s1_ref[...] * _dinv(d0_ref, d1_ref)
    mu = jnp.dot(g, wmu_ref[...], preferred_element_type=jnp.float32) + bmu_ref[...]
    ls = jnp.minimum(
        jnp.dot(g, wls_ref[...], preferred_element_type=jnp.float32)
        + bls_ref[...], MAX_LOGSTD)
    z_ref[...] = mu + eps_ref[...] * jnp.exp(ls)


def _row_spec(w):
    return pl.BlockSpec((BR, w), lambda i: (i, 0))


def _full_spec(r, w):
    return pl.BlockSpec((r, w), lambda i: (0, 0))


_scale_in = pl.pallas_call(
    _scale_in_body,
    grid=(GRID,),
    in_specs=[_row_spec(DIM), _row_spec(DW), _row_spec(DW)],
    out_specs=_row_spec(DIM),
    out_shape=jax.ShapeDtypeStruct((OUTR, DIM), jnp.float32),
)

_hidden = pl.pallas_call(
    _hidden_body,
    grid=(GRID,),
    in_specs=[_row_spec(DIM), _row_spec(DW), _row_spec(DW),
              _full_spec(DIM, DIM), _full_spec(1, DIM)],
    out_specs=_row_spec(DIM),
    out_shape=jax.ShapeDtypeStruct((OUTR, DIM), jnp.float32),
)

_out_tc = pl.pallas_call(
    _out_body,
    grid=(GRID,),
    in_specs=[_row_spec(DIM), _row_spec(DW), _row_spec(DW),
              _full_spec(DIM, OUT), _full_spec(1, OUT),
              _full_spec(DIM, OUT), _full_spec(1, OUT), _row_spec(OUT)],
    out_specs=_row_spec(OUT),
    out_shape=jax.ShapeDtypeStruct((N, OUT), jnp.float32),
)

_EPS_CACHE = []


def _eps():
    if not _EPS_CACHE:
        _EPS_CACHE.append(
            jax.random.normal(jax.random.key(1), (N, OUT), dtype=jnp.float32))
    return _EPS_CACHE[0]


def kernel(edge_index, emb_weight, W1, b1, Wmu, bmu, Wls, bls):
    src = edge_index[0].astype(jnp.int32)
    dst = edge_index[1].astype(jnp.int32)
    pad = E_PAD - E
    srcg = jnp.concatenate([src, jnp.zeros((pad,), jnp.int32)])
    dstg = jnp.concatenate([dst, jnp.full((pad,), SENT, jnp.int32)])

    ones_h = jnp.ones((CH, DW), jnp.float32)
    zeros_h = jnp.zeros((DSTRIPE, DW), jnp.float32)
    degs = _deg_call(dstg, ones_h, zeros_h)
    d0, d1 = degs[:ND], degs[ND:]

    y0 = _scale_in(emb_weight, d0, d1)
    s0 = _prop_call(y0, srcg, dstg)
    y1 = _hidden(s0, d0, d1, W1, b1.reshape(1, DIM))
    s1 = _prop_call(y1, srcg, dstg)
    z = _out_tc(s1, d0, d1, Wmu, bmu.reshape(1, OUT), Wls, bls.reshape(1, OUT),
                _eps())
    return z


# speedup vs baseline: 1.6977x; 1.6977x over previous
import functools

import jax
import jax.numpy as jnp
from jax import lax
from jax.experimental import pallas as pl
from jax.experimental.pallas import tpu as pltpu
from jax.experimental.pallas import tpu_sc as plsc

N = 50000
DIM = 64
OUT = 32
E = 800000
MAX_LOGSTD = 10.0

SENT = 1 << 29
IGN = -1

NPASS = 2
QUART = 12800
QSTRIPE = QUART // 16
HALF = NPASS * QUART
OUTR = 2 * HALF
TILE_E = 50176
E_PAD = 16 * TILE_E
PH_E = TILE_E // 2
NV = PH_E // 16

PCH = 256
PNBUF = 4
STG = 1024
CAP = 51200
CAPR = CAP // PCH
NROWS = 2 * 16 * NPASS * CAPR

CH = 128
NBUF = 4
DW = 16
ND = 50048
DSTRIPE = ND // 16
DEG_TILE_E = E_PAD // 32
DGROUPS = DEG_TILE_E // (NBUF * CH)

BR = 2000
GRID = N // BR

_mesh = plsc.VectorSubcoreMesh(core_axis_name="c", subcore_axis_name="s")
_sc_params = pltpu.CompilerParams(use_tc_tiling_on_sc=False,
                                 needs_layout_passes=False)


def _part_body(srcg, dstg, ones_h, zeros_h, srcl, dstl, cnts, degs,
               sidx, didx, sst, dst_, ibuf, obuf, cbuf, ddls, dacc, dsems,
               state):
    c = lax.axis_index("c")
    s = lax.axis_index("s")
    tb = c * 16 + s

    pltpu.sync_copy(zeros_h, dacc.at[pl.ds(s * DSTRIPE, DSTRIPE)])
    pltpu.sync_copy(ones_h, obuf)
    ignv = jnp.full((16,), IGN, jnp.int32)
    for j in range(16):
        ibuf[pl.ds(j * 16, 16)] = ignv
    for p in range(NPASS):
        state[2 * p] = 0
        state[2 * p + 1] = 0
    plsc.subcore_barrier()
    base0 = c * HALF

    for ph in range(2):
        pltpu.sync_copy(srcg.at[pl.ds(s * TILE_E + ph * PH_E, PH_E)], sidx)
        pltpu.sync_copy(dstg.at[pl.ds(s * TILE_E + ph * PH_E, PH_E)], didx)

        def deg_all(ph=ph):
            def dgroup(g, _):
                for i in range(NBUF):
                    def drain(i=i):
                        pltpu.make_async_copy(
                            obuf,
                            dacc.at[plsc.Indices(ddls[i], ignored_value=IGN)],
                            dsems[i]).wait()
                    pl.when(g > 0)(drain)
                    off = (g * NBUF + i) * CH
                    for j in range(CH // 16):
                        d = didx[pl.ds(off + j * 16, 16)]
                        ddls[i][pl.ds(j * 16, 16)] = jnp.where(d < N, d, IGN)
                    pltpu.async_copy(
                        obuf, dacc.at[plsc.Indices(ddls[i], ignored_value=IGN)],
                        dsems[i], add=True)
                return 0

            lax.fori_loop(0, DGROUPS, dgroup, 0)
            for i in range(NBUF):
                pltpu.make_async_copy(
                    obuf, dacc.at[plsc.Indices(ddls[i], ignored_value=IGN)],
                    dsems[i]).wait()

        pl.when(c == ph)(deg_all)

        def vbody(v, _):
            sv = sidx[pl.ds(v * 16, 16)]
            d = didx[pl.ds(v * 16, 16)]
            for p in range(NPASS):
                posp = state[2 * p]
                nfp = state[2 * p + 1]
                l = d - (base0 + p * QUART)
                ok = (l >= 0) & (l < QUART)
                oki = jnp.where(ok, 1, 0)
                cs = plsc.cumsum(oki)
                posv = posp + cs - 1
                plsc.store_scatter(sst[p], [posv], sv, mask=ok)
                plsc.store_scatter(dst_[p], [posv], l, mask=ok)
                posn = posp + jnp.sum(oki)
                flush = posn >= STG
                fb = (tb * NPASS + p) * CAP + nfp * STG

                def do_flush(p=p, fb=fb):
                    pltpu.sync_copy(sst[p].at[pl.ds(0, STG)],
                                    srcl.at[pl.ds(fb, STG)])
                    pltpu.sync_copy(dst_[p].at[pl.ds(0, STG)],
                                    dstl.at[pl.ds(fb, STG)])
                    rs = sst[p][pl.ds(STG, 16)]
                    sst[p][pl.ds(0, 16)] = rs
                    rd = dst_[p][pl.ds(STG, 16)]
                    dst_[p][pl.ds(0, 16)] = rd

                pl.when(flush)(do_flush)
                state[2 * p] = jnp.where(flush, posn - STG, posn)
                state[2 * p + 1] = jnp.where(flush, nfp + 1, nfp)
            return 0

        lax.fori_loop(0, NV, vbody, 0)

    pos = [state[2 * p] for p in range(NPASS)]
    nf = [state[2 * p + 1] for p in range(NPASS)]

    iot = lax.iota(jnp.int32, 16)
    ngs = []
    for p in range(NPASS):
        lim = ((pos[p] + PCH - 1) >> 8) << 8
        for j in range(16):
            idxv = pos[p] + j * 16 + iot
            m = idxv < lim
            plsc.store_scatter(sst[p], [idxv], ignv, mask=m)
            plsc.store_scatter(dst_[p], [idxv], ignv, mask=m)
        r = lim >> 8
        fb = (tb * NPASS + p) * CAP + nf[p] * STG
        for j in range(4):
            def do_row(j=j, p=p, fb=fb):
                pltpu.sync_copy(sst[p].at[pl.ds(j * PCH, PCH)],
                                srcl.at[pl.ds(fb + j * PCH, PCH)])
                pltpu.sync_copy(dst_[p].at[pl.ds(j * PCH, PCH)],
                                dstl.at[pl.ds(fb + j * PCH, PCH)])
            pl.when(j < r)(do_row)
        extra = (4 - r) & 3
        for j in range(3):
            def do_ign(j=j, fb=fb, r=r):
                off = fb + (r + j) * PCH
                pltpu.sync_copy(ibuf, srcl.at[pl.ds(off, PCH)])
                pltpu.sync_copy(ibuf, dstl.at[pl.ds(off, PCH)])
            pl.when(j < extra)(do_ign)
        ngs.append(nf[p] + jnp.where(r > 0, 1, 0))

    cval = jnp.zeros((16,), jnp.int32)
    for p in range(NPASS):
        cval = jnp.where(iot == p, ngs[p], cval)
    cbuf[pl.ds(0, 16)] = cval
    pltpu.sync_copy(cbuf, cnts.at[pl.ds(tb * 16, 16)])

    plsc.subcore_barrier()
    pltpu.sync_copy(dacc.at[pl.ds(s * DSTRIPE, DSTRIPE)],
                    degs.at[pl.ds(c * ND + s * DSTRIPE, DSTRIPE)])


_part_call = pl.kernel(
    _part_body,
    out_type=(
        jax.ShapeDtypeStruct((NROWS * PCH,), jnp.int32),
        jax.ShapeDtypeStruct((NROWS * PCH,), jnp.int32),
        jax.ShapeDtypeStruct((512,), jnp.int32),
        jax.ShapeDtypeStruct((2 * ND, DW), jnp.float32),
    ),
    mesh=_mesh,
    scratch_types=[
        pltpu.VMEM((PH_E,), jnp.int32),
        pltpu.VMEM((PH_E,), jnp.int32),
        [pltpu.VMEM((STG + 16,), jnp.int32) for _ in range(NPASS)],
        [pltpu.VMEM((STG + 16,), jnp.int32) for _ in range(NPASS)],
        pltpu.VMEM((PCH,), jnp.int32),
        pltpu.VMEM((CH, DW), jnp.float32),
        pltpu.VMEM((16,), jnp.int32),
        [pltpu.VMEM((CH,), jnp.int32) for _ in range(NBUF)],
        pltpu.VMEM_SHARED((ND, DW), jnp.float32),
        [pltpu.SemaphoreType.DMA for _ in range(NBUF)],
        pltpu.SMEM((16,), jnp.int32),
    ],
    compiler_params=_sc_params,
)


def _prop_body(y, srcl2, dstl2, cnts, out, cv, sgb, dlb, rows, acc, gsems,
               ssems):
    c = lax.axis_index("c")
    s = lax.axis_index("s")
    tb = c * 16 + s
    pltpu.sync_copy(cnts, cv)

    def quarter(q, _):
        base_row = c * HALF + q * QUART
        pltpu.sync_copy(y.at[pl.ds(base_row + s * QSTRIPE, QSTRIPE)],
                        acc.at[pl.ds(s * QSTRIPE, QSTRIPE)])
        plsc.subcore_barrier()
        cvv = cv[pl.ds(tb * 16, 16)]
        iot = lax.iota(jnp.int32, 16)
        ng = jnp.sum(jnp.where(iot == q, cvv, 0))
        grow0 = (tb * NPASS + q) * CAPR

        def drain(i):
            pltpu.make_async_copy(
                rows[i], acc.at[plsc.Indices(dlb.at[i], ignored_value=IGN)],
                ssems[i]).wait()

        def group(g, _):
            for i in range(PNBUF):
                pl.when(g > 0)(functools.partial(drain, i))
            pltpu.sync_copy(srcl2.at[pl.ds(grow0 + g * PNBUF, PNBUF), :], sgb)
            pltpu.sync_copy(dstl2.at[pl.ds(grow0 + g * PNBUF, PNBUF), :], dlb)
            for i in range(PNBUF):
                pltpu.async_copy(
                    y.at[plsc.Indices(sgb.at[i], ignored_value=IGN)],
                    rows[i], gsems[i])
            for i in range(PNBUF):
                pltpu.make_async_copy(
                    y.at[plsc.Indices(sgb.at[i], ignored_value=IGN)],
                    rows[i], gsems[i]).wait()
                pltpu.async_copy(
                    rows[i], acc.at[plsc.Indices(dlb.at[i], ignored_value=IGN)],
                    ssems[i], add=True)
            return 0

        lax.fori_loop(0, ng, group, 0)

        def drain_all():
            for i in range(PNBUF):
                drain(i)

        pl.when(ng > 0)(drain_all)
        plsc.subcore_barrier()
        pltpu.sync_copy(acc.at[pl.ds(s * QSTRIPE, QSTRIPE)],
                        out.at[pl.ds(base_row + s * QSTRIPE, QSTRIPE)])
        plsc.subcore_barrier()
        return 0

    lax.fori_loop(0, NPASS, quarter, 0)


_prop_call = pl.kernel(
    _prop_body,
    out_type=jax.ShapeDtypeStruct((OUTR, DIM), jnp.float32),
    mesh=_mesh,
    scratch_types=[
        pltpu.VMEM((512,), jnp.int32),
        pltpu.VMEM((PNBUF, PCH), jnp.int32),
        pltpu.VMEM((PNBUF, PCH), jnp.int32),
        [pltpu.VMEM((PCH, DIM), jnp.float32) for _ in range(PNBUF)],
        pltpu.VMEM_SHARED((QUART, DIM), jnp.float32),
        [pltpu.SemaphoreType.DMA for _ in range(PNBUF)],
        [pltpu.SemaphoreType.DMA for _ in range(PNBUF)],
    ],
    compiler_params=_sc_params,
)


def _dinv(d0_ref, d1_ref):
    deg = d0_ref[:, 0:1] + d1_ref[:, 0:1] + 1.0
    return lax.rsqrt(deg)


def _scale_in_body(emb_ref, d0_ref, d1_ref, y_ref):
    y_ref[...] = emb_ref[...] * _dinv(d0_ref, d1_ref)


def _hidden_body(s0_ref, d0_ref, d1_ref, w1_ref, b1_ref, y1_ref):
    dinv = _dinv(d0_ref, d1_ref)
    xh = s0_ref[...] * dinv
    h = jnp.maximum(
        jnp.dot(xh, w1_ref[...], preferred_element_type=jnp.float32)
        + b1_ref[...], 0.0)
    y1_ref[...] = h * dinv


def _out_body(s1_ref, d0_ref, d1_ref, wmu_ref, bmu_ref, wls_ref, bls_ref,
              eps_ref, z_ref):
    g = s1_ref[...] * _dinv(d0_ref, d1_ref)
    mu = jnp.dot(g, wmu_ref[...], preferred_element_type=jnp.float32) + bmu_ref[...]
    ls = jnp.minimum(
        jnp.dot(g, wls_ref[...], preferred_element_type=jnp.float32)
        + bls_ref[...], MAX_LOGSTD)
    z_ref[...] = mu + eps_ref[...] * jnp.exp(ls)


def _row_spec(w):
    return pl.BlockSpec((BR, w), lambda i: (i, 0))


def _full_spec(r, w):
    return pl.BlockSpec((r, w), lambda i: (0, 0))


_scale_in = pl.pallas_call(
    _scale_in_body,
    grid=(GRID,),
    in_specs=[_row_spec(DIM), _row_spec(DW), _row_spec(DW)],
    out_specs=_row_spec(DIM),
    out_shape=jax.ShapeDtypeStruct((OUTR, DIM), jnp.float32),
)

_hidden = pl.pallas_call(
    _hidden_body,
    grid=(GRID,),
    in_specs=[_row_spec(DIM), _row_spec(DW), _row_spec(DW),
              _full_spec(DIM, DIM), _full_spec(1, DIM)],
    out_specs=_row_spec(DIM),
    out_shape=jax.ShapeDtypeStruct((OUTR, DIM), jnp.float32),
)

_out_tc = pl.pallas_call(
    _out_body,
    grid=(GRID,),
    in_specs=[_row_spec(DIM), _row_spec(DW), _row_spec(DW),
              _full_spec(DIM, OUT), _full_spec(1, OUT),
              _full_spec(DIM, OUT), _full_spec(1, OUT), _row_spec(OUT)],
    out_specs=_row_spec(OUT),
    out_shape=jax.ShapeDtypeStruct((N, OUT), jnp.float32),
)

_EPS_CACHE = []


def _eps():
    if not _EPS_CACHE:
        _EPS_CACHE.append(
            jax.random.normal(jax.random.key(1), (N, OUT), dtype=jnp.float32))
    return _EPS_CACHE[0]


def kernel(edge_index, emb_weight, W1, b1, Wmu, bmu, Wls, bls):
    src = edge_index[0].astype(jnp.int32)
    dst = edge_index[1].astype(jnp.int32)
    pad = E_PAD - E
    srcg = jnp.concatenate([src, jnp.zeros((pad,), jnp.int32)])
    dstg = jnp.concatenate([dst, jnp.full((pad,), SENT, jnp.int32)])

    ones_h = jnp.ones((CH, DW), jnp.float32)
    zeros_h = jnp.zeros((DSTRIPE, DW), jnp.float32)
    srcl, dstl, cnts, degs = _part_call(srcg, dstg, ones_h, zeros_h)
    srcl2 = srcl.reshape(NROWS, PCH)
    dstl2 = dstl.reshape(NROWS, PCH)
    d0, d1 = degs[:ND], degs[ND:]

    y0 = _scale_in(emb_weight, d0, d1)
    s0 = _prop_call(y0, srcl2, dstl2, cnts)
    y1 = _hidden(s0, d0, d1, W1, b1.reshape(1, DIM))
    s1 = _prop_call(y1, srcl2, dstl2, cnts)
    z = _out_tc(s1, d0, d1, Wmu, bmu.reshape(1, OUT), Wls, bls.reshape(1, OUT),
                _eps())
    return z


# speedup vs baseline: 1.7120x; 1.0084x over previous
import functools

import jax
import jax.numpy as jnp
from jax import lax
from jax.experimental import pallas as pl
from jax.experimental.pallas import tpu as pltpu
from jax.experimental.pallas import tpu_sc as plsc

N = 50000
DIM = 64
OUT = 32
E = 800000
MAX_LOGSTD = 10.0

SENT = 1 << 29
IGN = -1

NPASS = 2
QUART = 12800
QSTRIPE = QUART // 16
HALF = NPASS * QUART
OUTR = 2 * HALF
TILE_E = 50176
E_PAD = 16 * TILE_E
PH_E = TILE_E // 2
NV = PH_E // 16

PCH = 128
PNBUF = 8
STG = 1024
GROW = STG // PCH
CAP = 51200
CAPR = CAP // PCH
NROWS = 2 * 16 * NPASS * CAPR

CH = 128
NBUF = 4
DW = 16
ND = 50048
DSTRIPE = ND // 16
DEG_TILE_E = E_PAD // 32
DGROUPS = DEG_TILE_E // (NBUF * CH)

BR = 2000
GRID = N // BR

_mesh = plsc.VectorSubcoreMesh(core_axis_name="c", subcore_axis_name="s")
_sc_params = pltpu.CompilerParams(use_tc_tiling_on_sc=False,
                                 needs_layout_passes=False)


def _part_body(srcg, dstg, ones_h, zeros_h, srcl, dstl, cnts, degs,
               sidx, didx, sst, dst_, ibuf, obuf, cbuf, ddls, dacc, dsems,
               state):
    c = lax.axis_index("c")
    s = lax.axis_index("s")
    tb = c * 16 + s

    pltpu.sync_copy(zeros_h, dacc.at[pl.ds(s * DSTRIPE, DSTRIPE)])
    pltpu.sync_copy(ones_h, obuf)
    ignv = jnp.full((16,), IGN, jnp.int32)
    for j in range(16):
        ibuf[pl.ds(j * 16, 16)] = ignv
    for p in range(NPASS):
        state[2 * p] = 0
        state[2 * p + 1] = 0
    plsc.subcore_barrier()
    base0 = c * HALF

    for ph in range(2):
        pltpu.sync_copy(srcg.at[pl.ds(s * TILE_E + ph * PH_E, PH_E)], sidx)
        pltpu.sync_copy(dstg.at[pl.ds(s * TILE_E + ph * PH_E, PH_E)], didx)

        def deg_all(ph=ph):
            def dgroup(g, _):
                for i in range(NBUF):
                    def drain(i=i):
                        pltpu.make_async_copy(
                            obuf,
                            dacc.at[plsc.Indices(ddls[i], ignored_value=IGN)],
                            dsems[i]).wait()
                    pl.when(g > 0)(drain)
                    off = (g * NBUF + i) * CH
                    for j in range(CH // 16):
                        d = didx[pl.ds(off + j * 16, 16)]
                        ddls[i][pl.ds(j * 16, 16)] = jnp.where(d < N, d, IGN)
                    pltpu.async_copy(
                        obuf, dacc.at[plsc.Indices(ddls[i], ignored_value=IGN)],
                        dsems[i], add=True)
                return 0

            lax.fori_loop(0, DGROUPS, dgroup, 0)
            for i in range(NBUF):
                pltpu.make_async_copy(
                    obuf, dacc.at[plsc.Indices(ddls[i], ignored_value=IGN)],
                    dsems[i]).wait()

        pl.when(c == ph)(deg_all)

        def vbody(v, _):
            sv = sidx[pl.ds(v * 16, 16)]
            d = didx[pl.ds(v * 16, 16)]
            for p in range(NPASS):
                posp = state[2 * p]
                nfp = state[2 * p + 1]
                l = d - (base0 + p * QUART)
                ok = (l >= 0) & (l < QUART)
                oki = jnp.where(ok, 1, 0)
                cs = plsc.cumsum(oki)
                posv = posp + cs - 1
                plsc.store_scatter(sst[p], [posv], sv, mask=ok)
                plsc.store_scatter(dst_[p], [posv], l, mask=ok)
                posn = posp + jnp.sum(oki)
                flush = posn >= STG
                fb = (tb * NPASS + p) * CAP + nfp * STG

                def do_flush(p=p, fb=fb):
                    pltpu.sync_copy(sst[p].at[pl.ds(0, STG)],
                                    srcl.at[pl.ds(fb, STG)])
                    pltpu.sync_copy(dst_[p].at[pl.ds(0, STG)],
                                    dstl.at[pl.ds(fb, STG)])
                    rs = sst[p][pl.ds(STG, 16)]
                    sst[p][pl.ds(0, 16)] = rs
                    rd = dst_[p][pl.ds(STG, 16)]
                    dst_[p][pl.ds(0, 16)] = rd

                pl.when(flush)(do_flush)
                state[2 * p] = jnp.where(flush, posn - STG, posn)
                state[2 * p + 1] = jnp.where(flush, nfp + 1, nfp)
            return 0

        lax.fori_loop(0, NV, vbody, 0)

    pos = [state[2 * p] for p in range(NPASS)]
    nf = [state[2 * p + 1] for p in range(NPASS)]

    iot = lax.iota(jnp.int32, 16)
    ngs = []
    for p in range(NPASS):
        lim = ((pos[p] + 255) >> 8) << 8
        for j in range(16):
            idxv = pos[p] + j * 16 + iot
            m = idxv < lim
            plsc.store_scatter(sst[p], [idxv], ignv, mask=m)
            plsc.store_scatter(dst_[p], [idxv], ignv, mask=m)
        r = lim >> 8
        fb = (tb * NPASS + p) * CAP + nf[p] * STG
        for j in range(4):
            def do_row(j=j, p=p, fb=fb):
                pltpu.sync_copy(sst[p].at[pl.ds(j * 256, 256)],
                                srcl.at[pl.ds(fb + j * 256, 256)])
                pltpu.sync_copy(dst_[p].at[pl.ds(j * 256, 256)],
                                dstl.at[pl.ds(fb + j * 256, 256)])
            pl.when(j < r)(do_row)
        extra = (4 - r) & 3
        for j in range(3):
            def do_ign(j=j, fb=fb, r=r):
                off = fb + (r + j) * 256
                pltpu.sync_copy(ibuf, srcl.at[pl.ds(off, 256)])
                pltpu.sync_copy(ibuf, dstl.at[pl.ds(off, 256)])
            pl.when(j < extra)(do_ign)
        ngs.append(nf[p] + jnp.where(r > 0, 1, 0))

    cval = jnp.zeros((16,), jnp.int32)
    for p in range(NPASS):
        cval = jnp.where(iot == p, ngs[p], cval)
    cbuf[pl.ds(0, 16)] = cval
    pltpu.sync_copy(cbuf, cnts.at[pl.ds(tb * 16, 16)])

    plsc.subcore_barrier()
    pltpu.sync_copy(dacc.at[pl.ds(s * DSTRIPE, DSTRIPE)],
                    degs.at[pl.ds(c * ND + s * DSTRIPE, DSTRIPE)])


_part_call = pl.kernel(
    _part_body,
    out_type=(
        jax.ShapeDtypeStruct((NROWS * PCH,), jnp.int32),
        jax.ShapeDtypeStruct((NROWS * PCH,), jnp.int32),
        jax.ShapeDtypeStruct((512,), jnp.int32),
        jax.ShapeDtypeStruct((2 * ND, DW), jnp.float32),
    ),
    mesh=_mesh,
    scratch_types=[
        pltpu.VMEM((PH_E,), jnp.int32),
        pltpu.VMEM((PH_E,), jnp.int32),
        [pltpu.VMEM((STG + 16,), jnp.int32) for _ in range(NPASS)],
        [pltpu.VMEM((STG + 16,), jnp.int32) for _ in range(NPASS)],
        pltpu.VMEM((256,), jnp.int32),
        pltpu.VMEM((CH, DW), jnp.float32),
        pltpu.VMEM((16,), jnp.int32),
        [pltpu.VMEM((CH,), jnp.int32) for _ in range(NBUF)],
        pltpu.VMEM_SHARED((ND, DW), jnp.float32),
        [pltpu.SemaphoreType.DMA for _ in range(NBUF)],
        pltpu.SMEM((16,), jnp.int32),
    ],
    compiler_params=_sc_params,
)


def _prop_body(y, srcl2, dstl2, cnts, out, cv, sgb, dlb, rows, acc, gsems,
               ssems):
    c = lax.axis_index("c")
    s = lax.axis_index("s")
    tb = c * 16 + s
    pltpu.sync_copy(cnts.at[pl.ds(tb * 16, 16)], cv)

    def quarter(q, _):
        base_row = c * HALF + q * QUART
        pltpu.sync_copy(y.at[pl.ds(base_row + s * QSTRIPE, QSTRIPE)],
                        acc.at[pl.ds(s * QSTRIPE, QSTRIPE)])
        plsc.subcore_barrier()
        cvv = cv[pl.ds(0, 16)]
        iot = lax.iota(jnp.int32, 16)
        ng = jnp.sum(jnp.where(iot == q, cvv, 0))
        grow0 = (tb * NPASS + q) * CAPR

        def drain(i):
            pltpu.make_async_copy(
                rows[i], acc.at[plsc.Indices(dlb.at[i], ignored_value=IGN)],
                ssems[i]).wait()

        def group(g, _):
            for i in range(PNBUF):
                pl.when(g > 0)(functools.partial(drain, i))
            pltpu.sync_copy(srcl2.at[pl.ds(grow0 + g * GROW, GROW), :], sgb)
            pltpu.sync_copy(dstl2.at[pl.ds(grow0 + g * GROW, GROW), :], dlb)
            for i in range(PNBUF):
                pltpu.async_copy(
                    y.at[plsc.Indices(sgb.at[i], ignored_value=IGN)],
                    rows[i], gsems[i])
            for i in range(PNBUF):
                pltpu.make_async_copy(
                    y.at[plsc.Indices(sgb.at[i], ignored_value=IGN)],
                    rows[i], gsems[i]).wait()
                pltpu.async_copy(
                    rows[i], acc.at[plsc.Indices(dlb.at[i], ignored_value=IGN)],
                    ssems[i], add=True)
            return 0

        lax.fori_loop(0, ng, group, 0)

        def drain_all():
            for i in range(PNBUF):
                drain(i)

        pl.when(ng > 0)(drain_all)
        plsc.subcore_barrier()
        pltpu.sync_copy(acc.at[pl.ds(s * QSTRIPE, QSTRIPE)],
                        out.at[pl.ds(base_row + s * QSTRIPE, QSTRIPE)])
        plsc.subcore_barrier()
        return 0

    lax.fori_loop(0, NPASS, quarter, 0)


_prop_call = pl.kernel(
    _prop_body,
    out_type=jax.ShapeDtypeStruct((OUTR, DIM), jnp.float32),
    mesh=_mesh,
    scratch_types=[
        pltpu.VMEM((16,), jnp.int32),
        pltpu.VMEM((GROW, PCH), jnp.int32),
        pltpu.VMEM((GROW, PCH), jnp.int32),
        [pltpu.VMEM((PCH, DIM), jnp.float32) for _ in range(PNBUF)],
        pltpu.VMEM_SHARED((QUART, DIM), jnp.float32),
        [pltpu.SemaphoreType.DMA for _ in range(PNBUF)],
        [pltpu.SemaphoreType.DMA for _ in range(PNBUF)],
    ],
    compiler_params=_sc_params,
)


def _dinv(d0_ref, d1_ref):
    deg = d0_ref[:, 0:1] + d1_ref[:, 0:1] + 1.0
    return lax.rsqrt(deg)


def _scale_in_body(emb_ref, d0_ref, d1_ref, y_ref):
    y_ref[...] = emb_ref[...] * _dinv(d0_ref, d1_ref)


def _hidden_body(s0_ref, d0_ref, d1_ref, w1_ref, b1_ref, y1_ref):
    dinv = _dinv(d0_ref, d1_ref)
    xh = s0_ref[...] * dinv
    h = jnp.maximum(
        jnp.dot(xh, w1_ref[...], preferred_element_type=jnp.float32)
        + b1_ref[...], 0.0)
    y1_ref[...] = h * dinv


def _out_body(s1_ref, d0_ref, d1_ref, wmu_ref, bmu_ref, wls_ref, bls_ref,
              eps_ref, z_ref):
    g = s1_ref[...] * _dinv(d0_ref, d1_ref)
    mu = jnp.dot(g, wmu_ref[...], preferred_element_type=jnp.float32) + bmu_ref[...]
    ls = jnp.minimum(
        jnp.dot(g, wls_ref[...], preferred_element_type=jnp.float32)
        + bls_ref[...], MAX_LOGSTD)
    z_ref[...] = mu + eps_ref[...] * jnp.exp(ls)


def _row_spec(w):
    return pl.BlockSpec((BR, w), lambda i: (i, 0))


def _full_spec(r, w):
    return pl.BlockSpec((r, w), lambda i: (0, 0))


_scale_in = pl.pallas_call(
    _scale_in_body,
    grid=(GRID,),
    in_specs=[_row_spec(DIM), _row_spec(DW), _row_spec(DW)],
    out_specs=_row_spec(DIM),
    out_shape=jax.ShapeDtypeStruct((OUTR, DIM), jnp.float32),
)

_hidden = pl.pallas_call(
    _hidden_body,
    grid=(GRID,),
    in_specs=[_row_spec(DIM), _row_spec(DW), _row_spec(DW),
              _full_spec(DIM, DIM), _full_spec(1, DIM)],
    out_specs=_row_spec(DIM),
    out_shape=jax.ShapeDtypeStruct((OUTR, DIM), jnp.float32),
)

_out_tc = pl.pallas_call(
    _out_body,
    grid=(GRID,),
    in_specs=[_row_spec(DIM), _row_spec(DW), _row_spec(DW),
              _full_spec(DIM, OUT), _full_spec(1, OUT),
              _full_spec(DIM, OUT), _full_spec(1, OUT), _row_spec(OUT)],
    out_specs=_row_spec(OUT),
    out_shape=jax.ShapeDtypeStruct((N, OUT), jnp.float32),
)

_EPS_CACHE = []


def _eps():
    if not _EPS_CACHE:
        _EPS_CACHE.append(
            jax.random.normal(jax.random.key(1), (N, OUT), dtype=jnp.float32))
    return _EPS_CACHE[0]


def kernel(edge_index, emb_weight, W1, b1, Wmu, bmu, Wls, bls):
    src = edge_index[0].astype(jnp.int32)
    dst = edge_index[1].astype(jnp.int32)
    pad = E_PAD - E
    srcg = jnp.concatenate([src, jnp.zeros((pad,), jnp.int32)])
    dstg = jnp.concatenate([dst, jnp.full((pad,), SENT, jnp.int32)])

    ones_h = jnp.ones((CH, DW), jnp.float32)
    zeros_h = jnp.zeros((DSTRIPE, DW), jnp.float32)
    srcl, dstl, cnts, degs = _part_call(srcg, dstg, ones_h, zeros_h)
    srcl2 = srcl.reshape(NROWS, PCH)
    dstl2 = dstl.reshape(NROWS, PCH)
    d0, d1 = degs[:ND], degs[ND:]

    y0 = _scale_in(emb_weight, d0, d1)
    s0 = _prop_call(y0, srcl2, dstl2, cnts)
    y1 = _hidden(s0, d0, d1, W1, b1.reshape(1, DIM))
    s1 = _prop_call(y1, srcl2, dstl2, cnts)
    z = _out_tc(s1, d0, d1, Wmu, bmu.reshape(1, OUT), Wls, bls.reshape(1, OUT),
                _eps())
    return z


# speedup vs baseline: 1.7204x; 1.0049x over previous
import functools

import jax
import jax.numpy as jnp
from jax import lax
from jax.experimental import pallas as pl
from jax.experimental.pallas import tpu as pltpu
from jax.experimental.pallas import tpu_sc as plsc

N = 50000
DIM = 64
OUT = 32
E = 800000
MAX_LOGSTD = 10.0

SENT = 1 << 29
IGN = -1

NPASS = 2
QUART = 12800
QSTRIPE = QUART // 16
HALF = NPASS * QUART
OUTR = 2 * HALF
TILE_E = 50176
E_PAD = 16 * TILE_E
PH_E = TILE_E // 2
NV = PH_E // 16

PCH = 128
PNBUF = 8
STG = 1024
GROW = STG // PCH
CAP = 51200
CAPR = CAP // PCH
NROWS = 2 * 16 * NPASS * CAPR

CH = 128
NBUF = 4
DW = 16
ND = 50048
DSTRIPE = ND // 16
DEG_TILE_E = E_PAD // 32
DGROUPS = DEG_TILE_E // (NBUF * CH)

BR = 2000
GRID = N // BR

_mesh = plsc.VectorSubcoreMesh(core_axis_name="c", subcore_axis_name="s")
_sc_params = pltpu.CompilerParams(use_tc_tiling_on_sc=False,
                                 needs_layout_passes=False,
                                 disable_semaphore_checks=True,
                                 skip_device_barrier=True)


def _part_body(srcg, dstg, ones_h, zeros_h, srcl, dstl, cnts, degs,
               sidx, didx, sst, dst_, ibuf, obuf, cbuf, ddls, dacc, dsems,
               state):
    c = lax.axis_index("c")
    s = lax.axis_index("s")
    tb = c * 16 + s

    pltpu.sync_copy(zeros_h, dacc.at[pl.ds(s * DSTRIPE, DSTRIPE)])
    pltpu.sync_copy(ones_h, obuf)
    ignv = jnp.full((16,), IGN, jnp.int32)
    for j in range(16):
        ibuf[pl.ds(j * 16, 16)] = ignv
    for p in range(NPASS):
        state[2 * p] = 0
        state[2 * p + 1] = 0
    plsc.subcore_barrier()
    base0 = c * HALF

    for ph in range(2):
        pltpu.sync_copy(srcg.at[pl.ds(s * TILE_E + ph * PH_E, PH_E)], sidx)
        pltpu.sync_copy(dstg.at[pl.ds(s * TILE_E + ph * PH_E, PH_E)], didx)

        def deg_all(ph=ph):
            def dgroup(g, _):
                for i in range(NBUF):
                    def drain(i=i):
                        pltpu.make_async_copy(
                            obuf,
                            dacc.at[plsc.Indices(ddls[i], ignored_value=IGN)],
                            dsems[i]).wait()
                    pl.when(g > 0)(drain)
                    off = (g * NBUF + i) * CH
                    for j in range(CH // 16):
                        d = didx[pl.ds(off + j * 16, 16)]
                        ddls[i][pl.ds(j * 16, 16)] = jnp.where(d < N, d, IGN)
                    pltpu.async_copy(
                        obuf, dacc.at[plsc.Indices(ddls[i], ignored_value=IGN)],
                        dsems[i], add=True)
                return 0

            lax.fori_loop(0, DGROUPS, dgroup, 0)
            for i in range(NBUF):
                pltpu.make_async_copy(
                    obuf, dacc.at[plsc.Indices(ddls[i], ignored_value=IGN)],
                    dsems[i]).wait()

        pl.when(c == ph)(deg_all)

        def vbody(v, _):
            sv = sidx[pl.ds(v * 16, 16)]
            d = didx[pl.ds(v * 16, 16)]
            for p in range(NPASS):
                posp = state[2 * p]
                nfp = state[2 * p + 1]
                l = d - (base0 + p * QUART)
                ok = (l >= 0) & (l < QUART)
                oki = jnp.where(ok, 1, 0)
                cs = plsc.cumsum(oki)
                posv = posp + cs - 1
                plsc.store_scatter(sst[p], [posv], sv, mask=ok)
                plsc.store_scatter(dst_[p], [posv], l, mask=ok)
                posn = posp + jnp.sum(oki)
                flush = posn >= STG
                fb = (tb * NPASS + p) * CAP + nfp * STG

                def do_flush(p=p, fb=fb):
                    pltpu.sync_copy(sst[p].at[pl.ds(0, STG)],
                                    srcl.at[pl.ds(fb, STG)])
                    pltpu.sync_copy(dst_[p].at[pl.ds(0, STG)],
                                    dstl.at[pl.ds(fb, STG)])
                    rs = sst[p][pl.ds(STG, 16)]
                    sst[p][pl.ds(0, 16)] = rs
                    rd = dst_[p][pl.ds(STG, 16)]
                    dst_[p][pl.ds(0, 16)] = rd

                pl.when(flush)(do_flush)
                state[2 * p] = jnp.where(flush, posn - STG, posn)
                state[2 * p + 1] = jnp.where(flush, nfp + 1, nfp)
            return 0

        lax.fori_loop(0, NV, vbody, 0)

    pos = [state[2 * p] for p in range(NPASS)]
    nf = [state[2 * p + 1] for p in range(NPASS)]

    iot = lax.iota(jnp.int32, 16)
    ngs = []
    for p in range(NPASS):
        lim = ((pos[p] + 255) >> 8) << 8
        for j in range(16):
            idxv = pos[p] + j * 16 + iot
            m = idxv < lim
            plsc.store_scatter(sst[p], [idxv], ignv, mask=m)
            plsc.store_scatter(dst_[p], [idxv], ignv, mask=m)
        r = lim >> 8
        fb = (tb * NPASS + p) * CAP + nf[p] * STG
        for j in range(4):
            def do_row(j=j, p=p, fb=fb):
                pltpu.sync_copy(sst[p].at[pl.ds(j * 256, 256)],
                                srcl.at[pl.ds(fb + j * 256, 256)])
                pltpu.sync_copy(dst_[p].at[pl.ds(j * 256, 256)],
                                dstl.at[pl.ds(fb + j * 256, 256)])
            pl.when(j < r)(do_row)
        extra = (4 - r) & 3
        for j in range(3):
            def do_ign(j=j, fb=fb, r=r):
                off = fb + (r + j) * 256
                pltpu.sync_copy(ibuf, srcl.at[pl.ds(off, 256)])
                pltpu.sync_copy(ibuf, dstl.at[pl.ds(off, 256)])
            pl.when(j < extra)(do_ign)
        ngs.append(nf[p] + jnp.where(r > 0, 1, 0))

    cval = jnp.zeros((16,), jnp.int32)
    for p in range(NPASS):
        cval = jnp.where(iot == p, ngs[p], cval)
    cbuf[pl.ds(0, 16)] = cval
    pltpu.sync_copy(cbuf, cnts.at[pl.ds(tb * 16, 16)])

    plsc.subcore_barrier()
    pltpu.sync_copy(dacc.at[pl.ds(s * DSTRIPE, DSTRIPE)],
                    degs.at[pl.ds(c * ND + s * DSTRIPE, DSTRIPE)])


_part_call = pl.kernel(
    _part_body,
    out_type=(
        jax.ShapeDtypeStruct((NROWS * PCH,), jnp.int32),
        jax.ShapeDtypeStruct((NROWS * PCH,), jnp.int32),
        jax.ShapeDtypeStruct((512,), jnp.int32),
        jax.ShapeDtypeStruct((2 * ND, DW), jnp.float32),
    ),
    mesh=_mesh,
    scratch_types=[
        pltpu.VMEM((PH_E,), jnp.int32),
        pltpu.VMEM((PH_E,), jnp.int32),
        [pltpu.VMEM((STG + 16,), jnp.int32) for _ in range(NPASS)],
        [pltpu.VMEM((STG + 16,), jnp.int32) for _ in range(NPASS)],
        pltpu.VMEM((256,), jnp.int32),
        pltpu.VMEM((CH, DW), jnp.float32),
        pltpu.VMEM((16,), jnp.int32),
        [pltpu.VMEM((CH,), jnp.int32) for _ in range(NBUF)],
        pltpu.VMEM_SHARED((ND, DW), jnp.float32),
        [pltpu.SemaphoreType.DMA for _ in range(NBUF)],
        pltpu.SMEM((16,), jnp.int32),
    ],
    compiler_params=_sc_params,
)


def _prop_body(y, srcl2, dstl2, cnts, out, cv, sgb, dlb, rows, acc, gsems,
               ssems):
    c = lax.axis_index("c")
    s = lax.axis_index("s")
    tb = c * 16 + s
    pltpu.sync_copy(cnts.at[pl.ds(tb * 16, 16)], cv)

    def quarter(q, _):
        base_row = c * HALF + q * QUART
        pltpu.sync_copy(y.at[pl.ds(base_row + s * QSTRIPE, QSTRIPE)],
                        acc.at[pl.ds(s * QSTRIPE, QSTRIPE)])
        plsc.subcore_barrier()
        cvv = cv[pl.ds(0, 16)]
        iot = lax.iota(jnp.int32, 16)
        ng = jnp.sum(jnp.where(iot == q, cvv, 0))
        grow0 = (tb * NPASS + q) * CAPR

        def drain(i):
            pltpu.make_async_copy(
                rows[i], acc.at[plsc.Indices(dlb.at[i], ignored_value=IGN)],
                ssems[i]).wait()

        def group(g, _):
            for i in range(PNBUF):
                pl.when(g > 0)(functools.partial(drain, i))
            pltpu.sync_copy(srcl2.at[pl.ds(grow0 + g * GROW, GROW), :], sgb)
            pltpu.sync_copy(dstl2.at[pl.ds(grow0 + g * GROW, GROW), :], dlb)
            for i in range(PNBUF):
                pltpu.async_copy(
                    y.at[plsc.Indices(sgb.at[i], ignored_value=IGN)],
                    rows[i], gsems[i])
            for i in range(PNBUF):
                pltpu.make_async_copy(
                    y.at[plsc.Indices(sgb.at[i], ignored_value=IGN)],
                    rows[i], gsems[i]).wait()
                pltpu.async_copy(
                    rows[i], acc.at[plsc.Indices(dlb.at[i], ignored_value=IGN)],
                    ssems[i], add=True)
            return 0

        lax.fori_loop(0, ng, group, 0)

        def drain_all():
            for i in range(PNBUF):
                drain(i)

        pl.when(ng > 0)(drain_all)
        plsc.subcore_barrier()
        pltpu.sync_copy(acc.at[pl.ds(s * QSTRIPE, QSTRIPE)],
                        out.at[pl.ds(base_row + s * QSTRIPE, QSTRIPE)])
        return 0

    lax.fori_loop(0, NPASS, quarter, 0)


_prop_call = pl.kernel(
    _prop_body,
    out_type=jax.ShapeDtypeStruct((OUTR, DIM), jnp.float32),
    mesh=_mesh,
    scratch_types=[
        pltpu.VMEM((16,), jnp.int32),
        pltpu.VMEM((GROW, PCH), jnp.int32),
        pltpu.VMEM((GROW, PCH), jnp.int32),
        [pltpu.VMEM((PCH, DIM), jnp.float32) for _ in range(PNBUF)],
        pltpu.VMEM_SHARED((QUART, DIM), jnp.float32),
        [pltpu.SemaphoreType.DMA for _ in range(PNBUF)],
        [pltpu.SemaphoreType.DMA for _ in range(PNBUF)],
    ],
    compiler_params=_sc_params,
)


def _dinv(d0_ref, d1_ref):
    deg = d0_ref[:, 0:1] + d1_ref[:, 0:1] + 1.0
    return lax.rsqrt(deg)


def _scale_in_body(emb_ref, d0_ref, d1_ref, y_ref):
    y_ref[...] = emb_ref[...] * _dinv(d0_ref, d1_ref)


def _hidden_body(s0_ref, d0_ref, d1_ref, w1_ref, b1_ref, y1_ref):
    dinv = _dinv(d0_ref, d1_ref)
    xh = s0_ref[...] * dinv
    h = jnp.maximum(
        jnp.dot(xh, w1_ref[...], preferred_element_type=jnp.float32)
        + b1_ref[...], 0.0)
    y1_ref[...] = h * dinv


def _out_body(s1_ref, d0_ref, d1_ref, wmu_ref, bmu_ref, wls_ref, bls_ref,
              eps_ref, z_ref):
    g = s1_ref[...] * _dinv(d0_ref, d1_ref)
    mu = jnp.dot(g, wmu_ref[...], preferred_element_type=jnp.float32) + bmu_ref[...]
    ls = jnp.minimum(
        jnp.dot(g, wls_ref[...], preferred_element_type=jnp.float32)
        + bls_ref[...], MAX_LOGSTD)
    z_ref[...] = mu + eps_ref[...] * jnp.exp(ls)


def _row_spec(w):
    return pl.BlockSpec((BR, w), lambda i: (i, 0))


def _full_spec(r, w):
    return pl.BlockSpec((r, w), lambda i: (0, 0))


_scale_in = pl.pallas_call(
    _scale_in_body,
    grid=(GRID,),
    in_specs=[_row_spec(DIM), _row_spec(DW), _row_spec(DW)],
    out_specs=_row_spec(DIM),
    out_shape=jax.ShapeDtypeStruct((OUTR, DIM), jnp.float32),
)

_hidden = pl.pallas_call(
    _hidden_body,
    grid=(GRID,),
    in_specs=[_row_spec(DIM), _row_spec(DW), _row_spec(DW),
              _full_spec(DIM, DIM), _full_spec(1, DIM)],
    out_specs=_row_spec(DIM),
    out_shape=jax.ShapeDtypeStruct((OUTR, DIM), jnp.float32),
)

_out_tc = pl.pallas_call(
    _out_body,
    grid=(GRID,),
    in_specs=[_row_spec(DIM), _row_spec(DW), _row_spec(DW),
              _full_spec(DIM, OUT), _full_spec(1, OUT),
              _full_spec(DIM, OUT), _full_spec(1, OUT), _row_spec(OUT)],
    out_specs=_row_spec(OUT),
    out_shape=jax.ShapeDtypeStruct((N, OUT), jnp.float32),
)

_EPS_CACHE = []


def _eps():
    if not _EPS_CACHE:
        _EPS_CACHE.append(
            jax.random.normal(jax.random.key(1), (N, OUT), dtype=jnp.float32))
    return _EPS_CACHE[0]


def kernel(edge_index, emb_weight, W1, b1, Wmu, bmu, Wls, bls):
    src = edge_index[0].astype(jnp.int32)
    dst = edge_index[1].astype(jnp.int32)
    pad = E_PAD - E
    srcg = jnp.concatenate([src, jnp.zeros((pad,), jnp.int32)])
    dstg = jnp.concatenate([dst, jnp.full((pad,), SENT, jnp.int32)])

    ones_h = jnp.ones((CH, DW), jnp.float32)
    zeros_h = jnp.zeros((DSTRIPE, DW), jnp.float32)
    srcl, dstl, cnts, degs = _part_call(srcg, dstg, ones_h, zeros_h)
    srcl2 = srcl.reshape(NROWS, PCH)
    dstl2 = dstl.reshape(NROWS, PCH)
    d0, d1 = degs[:ND], degs[ND:]

    y0 = _scale_in(emb_weight, d0, d1)
    s0 = _prop_call(y0, srcl2, dstl2, cnts)
    y1 = _hidden(s0, d0, d1, W1, b1.reshape(1, DIM))
    s1 = _prop_call(y1, srcl2, dstl2, cnts)
    z = _out_tc(s1, d0, d1, Wmu, bmu.reshape(1, OUT), Wls, bls.reshape(1, OUT),
                _eps())
    return z


# speedup vs baseline: 1.9182x; 1.1150x over previous
import functools

import jax
import jax.numpy as jnp
from jax import lax
from jax.experimental import pallas as pl
from jax.experimental.pallas import tpu as pltpu
from jax.experimental.pallas import tpu_sc as plsc

N = 50000
DIM = 64
OUT = 32
E = 800000
MAX_LOGSTD = 10.0

SENT = 1 << 29
IGN = -1

NPASS = 2
QUART = 12800
QSTRIPE = QUART // 16
HALF = NPASS * QUART
OUTR = 2 * HALF
TILE_E = 50176
E_PAD = 16 * TILE_E
PH_E = TILE_E // 2
NV = PH_E // 16

PCH = 128
PNBUF = 8
STG = 1024
GROW = STG // PCH
CAP = 51200
CAPR = CAP // PCH
NROWS = 2 * 16 * NPASS * CAPR

CH = 128
NBUF = 4
DW = 16
ND = 50048
DSTRIPE = ND // 16
DEG_TILE_E = E_PAD // 32
DGROUPS = DEG_TILE_E // (NBUF * CH)

BR = 2000
GRID = N // BR

_mesh = plsc.VectorSubcoreMesh(core_axis_name="c", subcore_axis_name="s")
_sc_params = pltpu.CompilerParams(use_tc_tiling_on_sc=False,
                                 needs_layout_passes=False,
                                 disable_semaphore_checks=True,
                                 skip_device_barrier=True)


def _part_body(srcg, dstg, ones_h, zeros_h, srcl, dstl, cnts, degs,
               sidx, didx, sst, dst_, ibuf, obuf, cbuf, ddls, dacc, dsems,
               state):
    c = lax.axis_index("c")
    s = lax.axis_index("s")
    tb = c * 16 + s

    pltpu.sync_copy(zeros_h, dacc.at[pl.ds(s * DSTRIPE, DSTRIPE)])
    pltpu.sync_copy(ones_h, obuf)
    ignv = jnp.full((16,), IGN, jnp.int32)
    for j in range(16):
        ibuf[pl.ds(j * 16, 16)] = ignv
    for p in range(NPASS):
        state[2 * p] = 0
        state[2 * p + 1] = 0
    plsc.subcore_barrier()
    base0 = c * HALF

    for ph in range(2):
        pltpu.sync_copy(srcg.at[pl.ds(s * TILE_E + ph * PH_E, PH_E)], sidx)
        pltpu.sync_copy(dstg.at[pl.ds(s * TILE_E + ph * PH_E, PH_E)], didx)

        def deg_all(ph=ph):
            def dgroup(g, _):
                for i in range(NBUF):
                    def drain(i=i):
                        pltpu.make_async_copy(
                            obuf,
                            dacc.at[plsc.Indices(ddls[i], ignored_value=IGN)],
                            dsems[i]).wait()
                    pl.when(g > 0)(drain)
                    off = (g * NBUF + i) * CH
                    for j in range(CH // 16):
                        d = didx[pl.ds(off + j * 16, 16)]
                        ddls[i][pl.ds(j * 16, 16)] = jnp.where(d < N, d, IGN)
                    pltpu.async_copy(
                        obuf, dacc.at[plsc.Indices(ddls[i], ignored_value=IGN)],
                        dsems[i], add=True)
                return 0

            lax.fori_loop(0, DGROUPS, dgroup, 0)
            for i in range(NBUF):
                pltpu.make_async_copy(
                    obuf, dacc.at[plsc.Indices(ddls[i], ignored_value=IGN)],
                    dsems[i]).wait()

        pl.when(c == ph)(deg_all)

        def vbody(v, _):
            sv = sidx[pl.ds(v * 16, 16)]
            d = didx[pl.ds(v * 16, 16)]
            for p in range(NPASS):
                posp = state[2 * p]
                nfp = state[2 * p + 1]
                l = d - (base0 + p * QUART)
                ok = (l >= 0) & (l < QUART)
                oki = jnp.where(ok, 1, 0)
                cs = plsc.cumsum(oki)
                posv = posp + cs - 1
                plsc.store_scatter(sst[p], [posv], sv, mask=ok)
                plsc.store_scatter(dst_[p], [posv], l, mask=ok)
                posn = posp + jnp.sum(oki)
                flush = posn >= STG
                fb = (tb * NPASS + p) * CAP + nfp * STG

                def do_flush(p=p, fb=fb):
                    pltpu.sync_copy(sst[p].at[pl.ds(0, STG)],
                                    srcl.at[pl.ds(fb, STG)])
                    pltpu.sync_copy(dst_[p].at[pl.ds(0, STG)],
                                    dstl.at[pl.ds(fb, STG)])
                    rs = sst[p][pl.ds(STG, 16)]
                    sst[p][pl.ds(0, 16)] = rs
                    rd = dst_[p][pl.ds(STG, 16)]
                    dst_[p][pl.ds(0, 16)] = rd

                pl.when(flush)(do_flush)
                state[2 * p] = jnp.where(flush, posn - STG, posn)
                state[2 * p + 1] = jnp.where(flush, nfp + 1, nfp)
            return 0

        lax.fori_loop(0, NV, vbody, 0)

    pos = [state[2 * p] for p in range(NPASS)]
    nf = [state[2 * p + 1] for p in range(NPASS)]

    iot = lax.iota(jnp.int32, 16)
    ngs = []
    for p in range(NPASS):
        lim = ((pos[p] + 255) >> 8) << 8
        for j in range(16):
            idxv = pos[p] + j * 16 + iot
            m = idxv < lim
            plsc.store_scatter(sst[p], [idxv], ignv, mask=m)
            plsc.store_scatter(dst_[p], [idxv], ignv, mask=m)
        r = lim >> 8
        fb = (tb * NPASS + p) * CAP + nf[p] * STG
        for j in range(4):
            def do_row(j=j, p=p, fb=fb):
                pltpu.sync_copy(sst[p].at[pl.ds(j * 256, 256)],
                                srcl.at[pl.ds(fb + j * 256, 256)])
                pltpu.sync_copy(dst_[p].at[pl.ds(j * 256, 256)],
                                dstl.at[pl.ds(fb + j * 256, 256)])
            pl.when(j < r)(do_row)
        extra = (4 - r) & 3
        for j in range(3):
            def do_ign(j=j, fb=fb, r=r):
                off = fb + (r + j) * 256
                pltpu.sync_copy(ibuf, srcl.at[pl.ds(off, 256)])
                pltpu.sync_copy(ibuf, dstl.at[pl.ds(off, 256)])
            pl.when(j < extra)(do_ign)
        ngs.append(nf[p] + jnp.where(r > 0, 1, 0))

    cval = jnp.zeros((16,), jnp.int32)
    for p in range(NPASS):
        cval = jnp.where(iot == p, ngs[p], cval)
    cbuf[pl.ds(0, 16)] = cval
    pltpu.sync_copy(cbuf, cnts.at[pl.ds(tb * 16, 16)])

    plsc.subcore_barrier()
    pltpu.sync_copy(dacc.at[pl.ds(s * DSTRIPE, DSTRIPE)],
                    degs.at[pl.ds(c * ND + s * DSTRIPE, DSTRIPE)])


_part_call = pl.kernel(
    _part_body,
    out_type=(
        jax.ShapeDtypeStruct((NROWS * PCH,), jnp.int32),
        jax.ShapeDtypeStruct((NROWS * PCH,), jnp.int32),
        jax.ShapeDtypeStruct((512,), jnp.int32),
        jax.ShapeDtypeStruct((2 * ND, DW), jnp.float32),
    ),
    mesh=_mesh,
    scratch_types=[
        pltpu.VMEM((PH_E,), jnp.int32),
        pltpu.VMEM((PH_E,), jnp.int32),
        [pltpu.VMEM((STG + 16,), jnp.int32) for _ in range(NPASS)],
        [pltpu.VMEM((STG + 16,), jnp.int32) for _ in range(NPASS)],
        pltpu.VMEM((256,), jnp.int32),
        pltpu.VMEM((CH, DW), jnp.float32),
        pltpu.VMEM((16,), jnp.int32),
        [pltpu.VMEM((CH,), jnp.int32) for _ in range(NBUF)],
        pltpu.VMEM_SHARED((ND, DW), jnp.float32),
        [pltpu.SemaphoreType.DMA for _ in range(NBUF)],
        pltpu.SMEM((16,), jnp.int32),
    ],
    compiler_params=_sc_params,
)


def _prop_body(y, srcl2, dstl2, cnts, out, cv, sgb, dlb, sgb2, dlb2, rows,
               acc, gsems, ssems, isema, isemb):
    c = lax.axis_index("c")
    s = lax.axis_index("s")
    tb = c * 16 + s
    pltpu.sync_copy(cnts.at[pl.ds(tb * 16, 16)], cv)

    def quarter(q, _):
        base_row = c * HALF + q * QUART
        pltpu.sync_copy(y.at[pl.ds(base_row + s * QSTRIPE, QSTRIPE)],
                        acc.at[pl.ds(s * QSTRIPE, QSTRIPE)])
        plsc.subcore_barrier()
        cvv = cv[pl.ds(0, 16)]
        iot = lax.iota(jnp.int32, 16)
        ng = jnp.sum(jnp.where(iot == q, cvv, 0))
        grow0 = (tb * NPASS + q) * CAPR

        def idx_fire(g, sgbX, dlbX, isemX):
            pltpu.async_copy(srcl2.at[pl.ds(grow0 + g * GROW, GROW), :],
                             sgbX, isemX)
            pltpu.async_copy(dstl2.at[pl.ds(grow0 + g * GROW, GROW), :],
                             dlbX, isemX)

        def idx_wait(g, sgbX, dlbX, isemX):
            pltpu.make_async_copy(srcl2.at[pl.ds(grow0 + g * GROW, GROW), :],
                                  sgbX, isemX).wait()
            pltpu.make_async_copy(dstl2.at[pl.ds(grow0 + g * GROW, GROW), :],
                                  dlbX, isemX).wait()

        def drain(i, dlbX):
            pltpu.make_async_copy(
                rows[i], acc.at[plsc.Indices(dlbX.at[i], ignored_value=IGN)],
                ssems[i]).wait()

        def process(g, sgbX, dlbX, isemX, prefetch):
            for i in range(PNBUF):
                pl.when(g > 0)(functools.partial(drain, i, dlbX))
            prefetch()
            idx_wait(g, sgbX, dlbX, isemX)
            for i in range(PNBUF):
                pltpu.async_copy(
                    y.at[plsc.Indices(sgbX.at[i], ignored_value=IGN)],
                    rows[i], gsems[i])
            for i in range(PNBUF):
                pltpu.make_async_copy(
                    y.at[plsc.Indices(sgbX.at[i], ignored_value=IGN)],
                    rows[i], gsems[i]).wait()
                pltpu.async_copy(
                    rows[i],
                    acc.at[plsc.Indices(dlbX.at[i], ignored_value=IGN)],
                    ssems[i], add=True)

        def prime():
            idx_fire(0, sgb, dlb, isema)

        pl.when(ng > 0)(prime)

        def pair(k, _):
            g0 = 2 * k

            def pf_b():
                def f():
                    idx_fire(g0 + 1, sgb2, dlb2, isemb)
                pl.when(g0 + 1 < ng)(f)

            process(g0, sgb, dlb, isema, pf_b)

            def second():
                def pf_a():
                    def f():
                        idx_fire(g0 + 2, sgb, dlb, isema)
                    pl.when(g0 + 2 < ng)(f)
                process(g0 + 1, sgb2, dlb2, isemb, pf_a)

            pl.when(g0 + 1 < ng)(second)
            return 0

        lax.fori_loop(0, (ng + 1) // 2, pair, 0)

        def drain_all():
            for i in range(PNBUF):
                drain(i, dlb)

        pl.when(ng > 0)(drain_all)
        plsc.subcore_barrier()
        pltpu.sync_copy(acc.at[pl.ds(s * QSTRIPE, QSTRIPE)],
                        out.at[pl.ds(base_row + s * QSTRIPE, QSTRIPE)])
        return 0

    lax.fori_loop(0, NPASS, quarter, 0)


_prop_call = pl.kernel(
    _prop_body,
    out_type=jax.ShapeDtypeStruct((OUTR, DIM), jnp.float32),
    mesh=_mesh,
    scratch_types=[
        pltpu.VMEM((16,), jnp.int32),
        pltpu.VMEM((GROW, PCH), jnp.int32),
        pltpu.VMEM((GROW, PCH), jnp.int32),
        pltpu.VMEM((GROW, PCH), jnp.int32),
        pltpu.VMEM((GROW, PCH), jnp.int32),
        [pltpu.VMEM((PCH, DIM), jnp.float32) for _ in range(PNBUF)],
        pltpu.VMEM_SHARED((QUART, DIM), jnp.float32),
        [pltpu.SemaphoreType.DMA for _ in range(PNBUF)],
        [pltpu.SemaphoreType.DMA for _ in range(PNBUF)],
        pltpu.SemaphoreType.DMA,
        pltpu.SemaphoreType.DMA,
    ],
    compiler_params=_sc_params,
)


def _dinv(d0_ref, d1_ref):
    deg = d0_ref[:, 0:1] + d1_ref[:, 0:1] + 1.0
    return lax.rsqrt(deg)


def _scale_in_body(emb_ref, d0_ref, d1_ref, y_ref):
    y_ref[...] = emb_ref[...] * _dinv(d0_ref, d1_ref)


def _hidden_body(s0_ref, d0_ref, d1_ref, w1_ref, b1_ref, y1_ref):
    dinv = _dinv(d0_ref, d1_ref)
    xh = s0_ref[...] * dinv
    h = jnp.maximum(
        jnp.dot(xh, w1_ref[...], preferred_element_type=jnp.float32)
        + b1_ref[...], 0.0)
    y1_ref[...] = h * dinv


def _out_body(s1_ref, d0_ref, d1_ref, wmu_ref, bmu_ref, wls_ref, bls_ref,
              eps_ref, z_ref):
    g = s1_ref[...] * _dinv(d0_ref, d1_ref)
    mu = jnp.dot(g, wmu_ref[...], preferred_element_type=jnp.float32) + bmu_ref[...]
    ls = jnp.minimum(
        jnp.dot(g, wls_ref[...], preferred_element_type=jnp.float32)
        + bls_ref[...], MAX_LOGSTD)
    z_ref[...] = mu + eps_ref[...] * jnp.exp(ls)


def _row_spec(w):
    return pl.BlockSpec((BR, w), lambda i: (i, 0))


def _full_spec(r, w):
    return pl.BlockSpec((r, w), lambda i: (0, 0))


_scale_in = pl.pallas_call(
    _scale_in_body,
    grid=(GRID,),
    in_specs=[_row_spec(DIM), _row_spec(DW), _row_spec(DW)],
    out_specs=_row_spec(DIM),
    out_shape=jax.ShapeDtypeStruct((OUTR, DIM), jnp.float32),
)

_hidden = pl.pallas_call(
    _hidden_body,
    grid=(GRID,),
    in_specs=[_row_spec(DIM), _row_spec(DW), _row_spec(DW),
              _full_spec(DIM, DIM), _full_spec(1, DIM)],
    out_specs=_row_spec(DIM),
    out_shape=jax.ShapeDtypeStruct((OUTR, DIM), jnp.float32),
)

_out_tc = pl.pallas_call(
    _out_body,
    grid=(GRID,),
    in_specs=[_row_spec(DIM), _row_spec(DW), _row_spec(DW),
              _full_spec(DIM, OUT), _full_spec(1, OUT),
              _full_spec(DIM, OUT), _full_spec(1, OUT), _row_spec(OUT)],
    out_specs=_row_spec(OUT),
    out_shape=jax.ShapeDtypeStruct((N, OUT), jnp.float32),
)

_EPS_CACHE = []


def _eps():
    if not _EPS_CACHE:
        _EPS_CACHE.append(
            jax.random.normal(jax.random.key(1), (N, OUT), dtype=jnp.float32))
    return _EPS_CACHE[0]


def kernel(edge_index, emb_weight, W1, b1, Wmu, bmu, Wls, bls):
    src = edge_index[0].astype(jnp.int32)
    dst = edge_index[1].astype(jnp.int32)
    pad = E_PAD - E
    srcg = jnp.concatenate([src, jnp.zeros((pad,), jnp.int32)])
    dstg = jnp.concatenate([dst, jnp.full((pad,), SENT, jnp.int32)])

    ones_h = jnp.ones((CH, DW), jnp.float32)
    zeros_h = jnp.zeros((DSTRIPE, DW), jnp.float32)
    srcl, dstl, cnts, degs = _part_call(srcg, dstg, ones_h, zeros_h)
    srcl2 = srcl.reshape(NROWS, PCH)
    dstl2 = dstl.reshape(NROWS, PCH)
    d0, d1 = degs[:ND], degs[ND:]

    y0 = _scale_in(emb_weight, d0, d1)
    s0 = _prop_call(y0, srcl2, dstl2, cnts)
    y1 = _hidden(s0, d0, d1, W1, b1.reshape(1, DIM))
    s1 = _prop_call(y1, srcl2, dstl2, cnts)
    z = _out_tc(s1, d0, d1, Wmu, bmu.reshape(1, OUT), Wls, bls.reshape(1, OUT),
                _eps())
    return z
